# baseline (speedup 1.0000x reference)
import functools

import jax
import jax.numpy as jnp
from jax import lax
from jax.experimental import pallas as pl
from jax.experimental.pallas import tpu as pltpu
from jax.experimental.pallas import tpu_sc as plsc

H = 8
K = 32
D = 128
DH = D // H
SCALE = DH ** -0.5
P = 8192
B = 2


def _proj_body(feats_ref, xyz8_ref, wq_ref, wk_ref, wv_ref, wp1_ref,
               q_ref, tab_ref):
    f = feats_ref[...]
    q_ref[...] = jnp.dot(f, wq_ref[...], preferred_element_type=jnp.float32)
    kf = jnp.dot(f, wk_ref[...], preferred_element_type=jnp.float32)
    vf = jnp.dot(f, wv_ref[...], preferred_element_type=jnp.float32)
    t = jnp.dot(xyz8_ref[...], wp1_ref[...], preferred_element_type=jnp.float32)
    tab_ref[...] = jnp.concatenate([kf, vf, t], axis=1)


def _run_proj(feats2, xyz8f, Wq, Wk, Wv, Wp1p):
    BA = 1024
    n = feats2.shape[0]
    return pl.pallas_call(
        _proj_body,
        grid=(n // BA,),
        in_specs=[
            pl.BlockSpec((BA, D), lambda i: (i, 0)),
            pl.BlockSpec((BA, 8), lambda i: (i, 0)),
            pl.BlockSpec((D, D), lambda i: (0, 0)),
            pl.BlockSpec((D, D), lambda i: (0, 0)),
            pl.BlockSpec((D, D), lambda i: (0, 0)),
            pl.BlockSpec((8, D), lambda i: (0, 0)),
        ],
        out_specs=[
            pl.BlockSpec((BA, D), lambda i: (i, 0)),
            pl.BlockSpec((BA, 3 * D), lambda i: (i, 0)),
        ],
        out_shape=[
            jax.ShapeDtypeStruct((n, D), jnp.float32),
            jax.ShapeDtypeStruct((n, 3 * D), jnp.float32),
        ],
        compiler_params=pltpu.CompilerParams(
            dimension_semantics=("parallel",)),
    )(feats2, xyz8f, Wq, Wk, Wv, Wp1p)


def _knn_body(xyzq_ref, xyzT_ref, idx_ref):
    xq = xyzq_ref[...]
    xt = xyzT_ref[...]
    dot = jnp.dot(xq, xt, preferred_element_type=jnp.float32)
    x2q = jnp.sum(xq * xq, axis=1, keepdims=True)
    x2k = jnp.sum(xt * xt, axis=0, keepdims=True)
    d2 = x2q + x2k - 2.0 * dot
    iota = lax.broadcasted_iota(jnp.int32, d2.shape, 1)
    acc0 = jnp.zeros((d2.shape[0], K), jnp.int32)
    lane = lax.broadcasted_iota(jnp.int32, acc0.shape, 1)

    def step(tstep, carry):
        d2c, acc = carry
        m = jnp.min(d2c, axis=1, keepdims=True)
        j = jnp.min(jnp.where(d2c <= m, iota, jnp.int32(2 ** 30)),
                    axis=1, keepdims=True)
        d2c = jnp.where(iota == j, jnp.float32(3.0e38), d2c)
        acc = jnp.where(lane == tstep, j, acc)
        return d2c, acc

    _, acc = lax.fori_loop(0, K, step, (d2, acc0))
    idx_ref[...] = acc + pl.program_id(0) * P


def _run_knn(xyz8, xyzT8):
    QB = 128
    return pl.pallas_call(
        _knn_body,
        grid=(B, P // QB),
        in_specs=[
            pl.BlockSpec((None, QB, 8), lambda b, i: (b, i, 0)),
            pl.BlockSpec((None, 8, P), lambda b, i: (b, 0, 0)),
        ],
        out_specs=pl.BlockSpec((None, QB, K), lambda b, i: (b, i, 0)),
        out_shape=jax.ShapeDtypeStruct((B, P, K), jnp.int32),
        compiler_params=pltpu.CompilerParams(
            dimension_semantics=("parallel", "parallel")),
    )(xyz8, xyzT8)


_NC, _NS = 2, 16
_NW = _NC * _NS
_CHUNK = 128


def _make_sc_gather(R, Dw):
    rows_per_w = R // _NW
    nchunk = rows_per_w // _CHUNK
    mesh = plsc.VectorSubcoreMesh(core_axis_name="c", subcore_axis_name="s")

    @functools.partial(
        pl.kernel,
        mesh=mesh,
        out_type=jax.ShapeDtypeStruct((R, Dw), jnp.float32),
        scratch_types=[
            pltpu.VMEM((_CHUNK,), jnp.int32),
            pltpu.VMEM((_CHUNK,), jnp.int32),
            pltpu.VMEM((_CHUNK, Dw), jnp.float32),
            pltpu.VMEM((_CHUNK, Dw), jnp.float32),
            pltpu.SemaphoreType.DMA,
            pltpu.SemaphoreType.DMA,
        ],
    )
    def gather_k(table_hbm, idx_hbm, out_hbm, idx_a, idx_b, rows_a, rows_b,
                 sem_a, sem_b):
        wid = lax.axis_index("s") * _NC + lax.axis_index("c")
        base0 = wid * nchunk * _CHUNK
        pltpu.sync_copy(idx_hbm.at[pl.ds(base0, _CHUNK)], idx_a)
        pltpu.make_async_copy(table_hbm.at[idx_a], rows_a, sem_a).start()

        def body(i, _):
            even = lax.rem(i, 2) == 0
            base_n = base0 + (i + 1) * _CHUNK

            def do(idx_c, rows_c, sem_c, idx_n, rows_n, sem_n):
                @pl.when(i + 1 < nchunk)
                def _():
                    pltpu.sync_copy(idx_hbm.at[pl.ds(base_n, _CHUNK)], idx_n)
                    pltpu.make_async_copy(table_hbm.at[idx_n], rows_n,
                                          sem_n).start()

                pltpu.make_async_copy(table_hbm.at[idx_c], rows_c, sem_c).wait()
                pltpu.sync_copy(
                    rows_c, out_hbm.at[pl.ds(base0 + i * _CHUNK, _CHUNK)])

            @pl.when(even)
            def _():
                do(idx_a, rows_a, sem_a, idx_b, rows_b, sem_b)

            @pl.when(jnp.logical_not(even))
            def _():
                do(idx_b, rows_b, sem_b, idx_a, rows_a, sem_a)

            return 0

        lax.fori_loop(0, nchunk, body, 0)

    return gather_k


def _attn_body(g_ref, q_ref, tab_ref, wp2_ref, b2_ref, b1_ref,
               wproj_ref, bproj_ref, o_ref):
    QD = q_ref.shape[0]
    g = g_ref[...]
    k_nei = g[:, 0:D]
    v_nei = g[:, D:2 * D]
    t_nei = g[:, 2 * D:3 * D].reshape(QD, K, D)
    tq = tab_ref[:, 2 * D:3 * D]
    hid = jnp.maximum(t_nei - tq[:, None, :] + b1_ref[...], 0.0)
    pos = jnp.dot(hid.reshape(QD * K, D), wp2_ref[...],
                  preferred_element_type=jnp.float32) + b2_ref[...]
    q = q_ref[...]
    s = (k_nei + pos).reshape(QD, K, D) * q[:, None, :]
    ci = lax.broadcasted_iota(jnp.int32, (D, H), 0)
    hi = lax.broadcasted_iota(jnp.int32, (D, H), 1)
    eh = jnp.where(ci // DH == hi, jnp.float32(1), jnp.float32(0))
    logits = jnp.dot(s.reshape(QD * K, D), eh,
                     preferred_element_type=jnp.float32)
    logits = logits.reshape(QD, K, H) * SCALE
    m = jnp.max(logits, axis=1, keepdims=True)
    e = jnp.exp(logits - m)
    a = e / jnp.sum(e, axis=1, keepdims=True)
    cit = lax.broadcasted_iota(jnp.int32, (H, D), 0)
    hit = lax.broadcasted_iota(jnp.int32, (H, D), 1)
    eht = jnp.where(hit // DH == cit, jnp.float32(1), jnp.float32(0))
    a128 = jnp.dot(a.reshape(QD * K, H), eht,
                   preferred_element_type=jnp.float32).reshape(QD, K, D)
    o = jnp.sum(a128 * ((v_nei + pos).reshape(QD, K, D)), axis=1)
    o_ref[...] = jnp.dot(o, wproj_ref[...],
                         preferred_element_type=jnp.float32) + bproj_ref[...]


def _run_attn(g, q2, tab, Wp2, b2r, b1r, Wproj, bprojr):
    QD = 128
    n = q2.shape[0]
    return pl.pallas_call(
        _attn_body,
        grid=(n // QD,),
        in_specs=[
            pl.BlockSpec((QD * K, 3 * D), lambda i: (i, 0)),
            pl.BlockSpec((QD, D), lambda i: (i, 0)),
            pl.BlockSpec((QD, 3 * D), lambda i: (i, 0)),
            pl.BlockSpec((D, D), lambda i: (0, 0)),
            pl.BlockSpec((1, D), lambda i: (0, 0)),
            pl.BlockSpec((1, D), lambda i: (0, 0)),
            pl.BlockSpec((D, D), lambda i: (0, 0)),
            pl.BlockSpec((1, D), lambda i: (0, 0)),
        ],
        out_specs=pl.BlockSpec((QD, D), lambda i: (i, 0)),
        out_shape=jax.ShapeDtypeStruct((n, D), jnp.float32),
        compiler_params=pltpu.CompilerParams(
            dimension_semantics=("parallel",)),
    )(g, q2, tab, Wp2, b2r, b1r, Wproj, bprojr)


def kernel(xyz, feats, Wq, Wk, Wv, Wproj, bproj, Wp1, bp1, Wp2, bp2):
    pad = jnp.zeros((B, P, 5), xyz.dtype)
    xyz8 = jnp.concatenate([xyz, pad], axis=-1)
    xyzT8 = jnp.swapaxes(xyz8, 1, 2)
    Wp1p = jnp.concatenate([Wp1, jnp.zeros((5, D), Wp1.dtype)], axis=0)

    feats2 = feats.reshape(B * P, D)
    xyz8f = xyz8.reshape(B * P, 8)

    q2, tab = _run_proj(feats2, xyz8f, Wq, Wk, Wv, Wp1p)
    idx = _run_knn(xyz8, xyzT8)
    g = _make_sc_gather(B * P * K, 3 * D)(tab, idx.reshape(B * P * K))
    out = _run_attn(g, q2, tab, Wp2, bp2.reshape(1, D), bp1.reshape(1, D),
                    Wproj, bproj.reshape(1, D))
    return out.reshape(B, P, D)


# speedup vs baseline: 1.4244x; 1.4244x over previous
import functools

import jax
import jax.numpy as jnp
from jax import lax
from jax.experimental import pallas as pl
from jax.experimental.pallas import tpu as pltpu
from jax.experimental.pallas import tpu_sc as plsc

H = 8
K = 32
D = 128
DH = D // H
SCALE = DH ** -0.5
P = 8192
B = 2


def _proj_body(feats_ref, xyz8_ref, wq_ref, wk_ref, wv_ref, wp1_ref,
               q_ref, tab_ref):
    f = feats_ref[...]
    q_ref[...] = jnp.dot(f, wq_ref[...], preferred_element_type=jnp.float32)
    kf = jnp.dot(f, wk_ref[...], preferred_element_type=jnp.float32)
    vf = jnp.dot(f, wv_ref[...], preferred_element_type=jnp.float32)
    t = jnp.dot(xyz8_ref[...], wp1_ref[...], preferred_element_type=jnp.float32)
    tab_ref[...] = jnp.concatenate([kf, vf, t], axis=1)


def _run_proj(feats2, xyz8f, Wq, Wk, Wv, Wp1p):
    BA = 1024
    n = feats2.shape[0]
    return pl.pallas_call(
        _proj_body,
        grid=(n // BA,),
        in_specs=[
            pl.BlockSpec((BA, D), lambda i: (i, 0)),
            pl.BlockSpec((BA, 8), lambda i: (i, 0)),
            pl.BlockSpec((D, D), lambda i: (0, 0)),
            pl.BlockSpec((D, D), lambda i: (0, 0)),
            pl.BlockSpec((D, D), lambda i: (0, 0)),
            pl.BlockSpec((8, D), lambda i: (0, 0)),
        ],
        out_specs=[
            pl.BlockSpec((BA, D), lambda i: (i, 0)),
            pl.BlockSpec((BA, 3 * D), lambda i: (i, 0)),
        ],
        out_shape=[
            jax.ShapeDtypeStruct((n, D), jnp.float32),
            jax.ShapeDtypeStruct((n, 3 * D), jnp.float32),
        ],
        compiler_params=pltpu.CompilerParams(
            dimension_semantics=("parallel",)),
    )(feats2, xyz8f, Wq, Wk, Wv, Wp1p)


G = 32
NG = P // G


def _knn_groups_body(xyzq_ref, xyzT_ref, gid_ref):
    xq = xyzq_ref[...]
    xt = xyzT_ref[...]
    dot = jnp.dot(xq, xt, preferred_element_type=jnp.float32)
    x2q = jnp.sum(xq * xq, axis=1, keepdims=True)
    x2k = jnp.sum(xt * xt, axis=0, keepdims=True)
    d2 = x2q + x2k - 2.0 * dot
    QB = d2.shape[0]
    gmin = jnp.min(d2.reshape(QB, NG, G), axis=2)
    giota = lax.broadcasted_iota(jnp.int32, gmin.shape, 1)
    acc0 = jnp.zeros((QB, K), jnp.int32)
    lane = lax.broadcasted_iota(jnp.int32, acc0.shape, 1)

    def step(tstep, carry):
        gm, acc = carry
        m = jnp.min(gm, axis=1, keepdims=True)
        j = jnp.min(jnp.where(gm <= m, giota, jnp.int32(2 ** 30)),
                    axis=1, keepdims=True)
        gm = jnp.where(giota == j, jnp.float32(3.0e38), gm)
        acc = jnp.where(lane == tstep, j, acc)
        return gm, acc

    _, acc = lax.fori_loop(0, K, step, (gmin, acc0))
    gid_ref[...] = acc + pl.program_id(0) * NG


def _run_knn_groups(xyz8, xyzT8):
    QB = 128
    return pl.pallas_call(
        _knn_groups_body,
        grid=(B, P // QB),
        in_specs=[
            pl.BlockSpec((None, QB, 8), lambda b, i: (b, i, 0)),
            pl.BlockSpec((None, 8, P), lambda b, i: (b, 0, 0)),
        ],
        out_specs=pl.BlockSpec((None, QB, K), lambda b, i: (b, i, 0)),
        out_shape=jax.ShapeDtypeStruct((B, P, K), jnp.int32),
        compiler_params=pltpu.CompilerParams(
            dimension_semantics=("parallel", "parallel")),
    )(xyz8, xyzT8)


def _knn_refine_body(g1_ref, gid_ref, xyzq_ref, idx_ref):
    QB = gid_ref.shape[0]
    gv = g1_ref[...].reshape(QB, K, 8, G)
    xq = xyzq_ref[...]
    diff = gv - xq[:, None, :, None]
    d2c = jnp.sum(diff * diff, axis=2).reshape(QB, K * G)
    gid = gid_ref[...]
    piota = lax.broadcasted_iota(jnp.int32, (QB, K, G), 2)
    cidx = (gid[:, :, None] * G + piota).reshape(QB, K * G)
    acc0 = jnp.zeros((QB, K), jnp.int32)
    lane = lax.broadcasted_iota(jnp.int32, acc0.shape, 1)

    def step(tstep, carry):
        d2f, acc = carry
        m = jnp.min(d2f, axis=1, keepdims=True)
        j = jnp.min(jnp.where(d2f <= m, cidx, jnp.int32(2 ** 30)),
                    axis=1, keepdims=True)
        d2f = jnp.where(cidx == j, jnp.float32(3.0e38), d2f)
        acc = jnp.where(lane == tstep, j, acc)
        return d2f, acc

    _, acc = lax.fori_loop(0, K, step, (d2c, acc0))
    idx_ref[...] = acc


def _run_knn_refine(g1, gid2, xyz8f):
    QB = 128
    n = xyz8f.shape[0]
    return pl.pallas_call(
        _knn_refine_body,
        grid=(n // QB,),
        in_specs=[
            pl.BlockSpec((QB * K, 8 * G), lambda i: (i, 0)),
            pl.BlockSpec((QB, K), lambda i: (i, 0)),
            pl.BlockSpec((QB, 8), lambda i: (i, 0)),
        ],
        out_specs=pl.BlockSpec((QB, K), lambda i: (i, 0)),
        out_shape=jax.ShapeDtypeStruct((n, K), jnp.int32),
        compiler_params=pltpu.CompilerParams(
            dimension_semantics=("parallel",)),
    )(g1, gid2, xyz8f)


_NC, _NS = 2, 16
_NW = _NC * _NS
_CHUNK = 128


def _make_sc_gather(R, Dw):
    rows_per_w = R // _NW
    nchunk = rows_per_w // _CHUNK
    mesh = plsc.VectorSubcoreMesh(core_axis_name="c", subcore_axis_name="s")

    @functools.partial(
        pl.kernel,
        mesh=mesh,
        out_type=jax.ShapeDtypeStruct((R, Dw), jnp.float32),
        scratch_types=[
            pltpu.VMEM((_CHUNK,), jnp.int32),
            pltpu.VMEM((_CHUNK,), jnp.int32),
            pltpu.VMEM((_CHUNK, Dw), jnp.float32),
            pltpu.VMEM((_CHUNK, Dw), jnp.float32),
            pltpu.SemaphoreType.DMA,
            pltpu.SemaphoreType.DMA,
        ],
    )
    def gather_k(table_hbm, idx_hbm, out_hbm, idx_a, idx_b, rows_a, rows_b,
                 sem_a, sem_b):
        wid = lax.axis_index("s") * _NC + lax.axis_index("c")
        base0 = wid * nchunk * _CHUNK
        pltpu.sync_copy(idx_hbm.at[pl.ds(base0, _CHUNK)], idx_a)
        pltpu.make_async_copy(table_hbm.at[idx_a], rows_a, sem_a).start()

        def body(i, _):
            even = lax.rem(i, 2) == 0
            base_n = base0 + (i + 1) * _CHUNK

            def do(idx_c, rows_c, sem_c, idx_n, rows_n, sem_n):
                @pl.when(i + 1 < nchunk)
                def _():
                    pltpu.sync_copy(idx_hbm.at[pl.ds(base_n, _CHUNK)], idx_n)
                    pltpu.make_async_copy(table_hbm.at[idx_n], rows_n,
                                          sem_n).start()

                pltpu.make_async_copy(table_hbm.at[idx_c], rows_c, sem_c).wait()
                pltpu.sync_copy(
                    rows_c, out_hbm.at[pl.ds(base0 + i * _CHUNK, _CHUNK)])

            @pl.when(even)
            def _():
                do(idx_a, rows_a, sem_a, idx_b, rows_b, sem_b)

            @pl.when(jnp.logical_not(even))
            def _():
                do(idx_b, rows_b, sem_b, idx_a, rows_a, sem_a)

            return 0

        lax.fori_loop(0, nchunk, body, 0)

    return gather_k


def _attn_body(g_ref, q_ref, tab_ref, wp2_ref, b2_ref, b1_ref,
               wproj_ref, bproj_ref, o_ref):
    QD = q_ref.shape[0]
    g = g_ref[...]
    k_nei = g[:, 0:D]
    v_nei = g[:, D:2 * D]
    t_nei = g[:, 2 * D:3 * D].reshape(QD, K, D)
    tq = tab_ref[:, 2 * D:3 * D]
    hid = jnp.maximum(t_nei - tq[:, None, :] + b1_ref[...], 0.0)
    pos = jnp.dot(hid.reshape(QD * K, D), wp2_ref[...],
                  preferred_element_type=jnp.float32) + b2_ref[...]
    q = q_ref[...]
    s = (k_nei + pos).reshape(QD, K, D) * q[:, None, :]
    ci = lax.broadcasted_iota(jnp.int32, (D, H), 0)
    hi = lax.broadcasted_iota(jnp.int32, (D, H), 1)
    eh = jnp.where(ci // DH == hi, jnp.float32(1), jnp.float32(0))
    logits = jnp.dot(s.reshape(QD * K, D), eh,
                     preferred_element_type=jnp.float32)
    logits = logits.reshape(QD, K, H) * SCALE
    m = jnp.max(logits, axis=1, keepdims=True)
    e = jnp.exp(logits - m)
    a = e / jnp.sum(e, axis=1, keepdims=True)
    cit = lax.broadcasted_iota(jnp.int32, (H, D), 0)
    hit = lax.broadcasted_iota(jnp.int32, (H, D), 1)
    eht = jnp.where(hit // DH == cit, jnp.float32(1), jnp.float32(0))
    a128 = jnp.dot(a.reshape(QD * K, H), eht,
                   preferred_element_type=jnp.float32).reshape(QD, K, D)
    o = jnp.sum(a128 * ((v_nei + pos).reshape(QD, K, D)), axis=1)
    o_ref[...] = jnp.dot(o, wproj_ref[...],
                         preferred_element_type=jnp.float32) + bproj_ref[...]


def _run_attn(g, q2, tab, Wp2, b2r, b1r, Wproj, bprojr):
    QD = 128
    n = q2.shape[0]
    return pl.pallas_call(
        _attn_body,
        grid=(n // QD,),
        in_specs=[
            pl.BlockSpec((QD * K, 3 * D), lambda i: (i, 0)),
            pl.BlockSpec((QD, D), lambda i: (i, 0)),
            pl.BlockSpec((QD, 3 * D), lambda i: (i, 0)),
            pl.BlockSpec((D, D), lambda i: (0, 0)),
            pl.BlockSpec((1, D), lambda i: (0, 0)),
            pl.BlockSpec((1, D), lambda i: (0, 0)),
            pl.BlockSpec((D, D), lambda i: (0, 0)),
            pl.BlockSpec((1, D), lambda i: (0, 0)),
        ],
        out_specs=pl.BlockSpec((QD, D), lambda i: (i, 0)),
        out_shape=jax.ShapeDtypeStruct((n, D), jnp.float32),
        compiler_params=pltpu.CompilerParams(
            dimension_semantics=("parallel",)),
    )(g, q2, tab, Wp2, b2r, b1r, Wproj, bprojr)


def kernel(xyz, feats, Wq, Wk, Wv, Wproj, bproj, Wp1, bp1, Wp2, bp2):
    pad = jnp.zeros((B, P, 5), xyz.dtype)
    xyz8 = jnp.concatenate([xyz, pad], axis=-1)
    xyzT8 = jnp.swapaxes(xyz8, 1, 2)
    Wp1p = jnp.concatenate([Wp1, jnp.zeros((5, D), Wp1.dtype)], axis=0)

    feats2 = feats.reshape(B * P, D)
    xyz8f = xyz8.reshape(B * P, 8)

    q2, tab = _run_proj(feats2, xyz8f, Wq, Wk, Wv, Wp1p)
    gid = _run_knn_groups(xyz8, xyzT8)
    gt = xyz8.reshape(B, NG, G, 8).transpose(0, 1, 3, 2).reshape(B * NG, 8 * G)
    g1 = _make_sc_gather(B * P * K, 8 * G)(gt, gid.reshape(B * P * K))
    idx = _run_knn_refine(g1, gid.reshape(B * P, K), xyz8f)
    g = _make_sc_gather(B * P * K, 3 * D)(tab, idx.reshape(B * P * K))
    out = _run_attn(g, q2, tab, Wp2, bp2.reshape(1, D), bp1.reshape(1, D),
                    Wproj, bproj.reshape(1, D))
    return out.reshape(B, P, D)


# speedup vs baseline: 1.7703x; 1.2428x over previous
import functools

import jax
import jax.numpy as jnp
from jax import lax
from jax.experimental import pallas as pl
from jax.experimental.pallas import tpu as pltpu
from jax.experimental.pallas import tpu_sc as plsc

H = 8
K = 32
D = 128
DH = D // H
SCALE = DH ** -0.5
P = 8192
B = 2


def _proj_body(feats_ref, xyz8_ref, wq_ref, wk_ref, wv_ref, wp1_ref,
               q_ref, tab_ref):
    f = feats_ref[...]
    q_ref[...] = jnp.dot(f, wq_ref[...], preferred_element_type=jnp.float32)
    kf = jnp.dot(f, wk_ref[...], preferred_element_type=jnp.float32)
    vf = jnp.dot(f, wv_ref[...], preferred_element_type=jnp.float32)
    t = jnp.dot(xyz8_ref[...], wp1_ref[...], preferred_element_type=jnp.float32)
    tab_ref[...] = jnp.concatenate([kf, vf, t], axis=1)


def _run_proj(feats2, xyz8f, Wq, Wk, Wv, Wp1p):
    BA = 1024
    n = feats2.shape[0]
    return pl.pallas_call(
        _proj_body,
        grid=(n // BA,),
        in_specs=[
            pl.BlockSpec((BA, D), lambda i: (i, 0)),
            pl.BlockSpec((BA, 8), lambda i: (i, 0)),
            pl.BlockSpec((D, D), lambda i: (0, 0)),
            pl.BlockSpec((D, D), lambda i: (0, 0)),
            pl.BlockSpec((D, D), lambda i: (0, 0)),
            pl.BlockSpec((8, D), lambda i: (0, 0)),
        ],
        out_specs=[
            pl.BlockSpec((BA, D), lambda i: (i, 0)),
            pl.BlockSpec((BA, 3 * D), lambda i: (i, 0)),
        ],
        out_shape=[
            jax.ShapeDtypeStruct((n, D), jnp.float32),
            jax.ShapeDtypeStruct((n, 3 * D), jnp.float32),
        ],
        compiler_params=pltpu.CompilerParams(
            dimension_semantics=("parallel",)),
    )(feats2, xyz8f, Wq, Wk, Wv, Wp1p)


G = 32
NG = P // G


def _knn_groups_body(xyzq_ref, xyzT_ref, gid_ref):
    xq = xyzq_ref[...]
    xt = xyzT_ref[...]
    dot = jnp.dot(xq, xt, preferred_element_type=jnp.float32)
    x2q = jnp.sum(xq * xq, axis=1, keepdims=True)
    x2k = jnp.sum(xt * xt, axis=0, keepdims=True)
    d2 = x2q + x2k - 2.0 * dot
    QB = d2.shape[0]
    gmin = jnp.min(d2.reshape(QB, NG, G), axis=2)
    giota = lax.broadcasted_iota(jnp.int32, gmin.shape, 1)
    acc0 = jnp.zeros((QB, K), jnp.int32)
    lane = lax.broadcasted_iota(jnp.int32, acc0.shape, 1)

    def step(tstep, carry):
        gm, acc = carry
        m = jnp.min(gm, axis=1, keepdims=True)
        j = jnp.min(jnp.where(gm <= m, giota, jnp.int32(2 ** 30)),
                    axis=1, keepdims=True)
        gm = jnp.where(giota == j, jnp.float32(3.0e38), gm)
        acc = jnp.where(lane == tstep, j, acc)
        return gm, acc

    _, acc = lax.fori_loop(0, K, step, (gmin, acc0))
    gid_ref[...] = acc + pl.program_id(0) * NG


def _run_knn_groups(xyz8, xyzT8):
    QB = 128
    return pl.pallas_call(
        _knn_groups_body,
        grid=(B, P // QB),
        in_specs=[
            pl.BlockSpec((None, QB, 8), lambda b, i: (b, i, 0)),
            pl.BlockSpec((None, 8, P), lambda b, i: (b, 0, 0)),
        ],
        out_specs=pl.BlockSpec((None, QB, K), lambda b, i: (b, i, 0)),
        out_shape=jax.ShapeDtypeStruct((B, P, K), jnp.int32),
        compiler_params=pltpu.CompilerParams(
            dimension_semantics=("parallel", "parallel")),
    )(xyz8, xyzT8)


def _knn_dist_body(g1_ref, xyzq_ref, d2_ref):
    QB = xyzq_ref.shape[0]
    g3 = g1_ref[...].reshape(QB, K, 4 * G)
    xq = xyzq_ref[...]
    acc = g3[:, :, 3 * G:4 * G]
    for c in range(3):
        xc = (-2.0 * xq[:, c])[:, None, None]
        acc = acc + g3[:, :, c * G:(c + 1) * G] * xc
    d2_ref[...] = acc.reshape(QB * K, G)


def _run_knn_dist(g1, xyz8f):
    QB = 128
    n = xyz8f.shape[0]
    return pl.pallas_call(
        _knn_dist_body,
        grid=(n // QB,),
        in_specs=[
            pl.BlockSpec((QB * K, 4 * G), lambda i: (i, 0)),
            pl.BlockSpec((QB, 8), lambda i: (i, 0)),
        ],
        out_specs=pl.BlockSpec((QB * K, G), lambda i: (i, 0)),
        out_shape=jax.ShapeDtypeStruct((n * K, G), jnp.float32),
        compiler_params=pltpu.CompilerParams(
            dimension_semantics=("parallel",)),
    )(g1, xyz8f)


def _knn_select_body(d2_ref, gid_ref, idx_ref):
    QB = gid_ref.shape[0]
    d2c = d2_ref[...]
    gidf = gid_ref[...].astype(jnp.float32)
    ki = lax.broadcasted_iota(jnp.int32, (K, K * G), 0)
    pi = lax.broadcasted_iota(jnp.int32, (K, K * G), 1)
    ef = jnp.where(pi // G == ki, jnp.float32(1), jnp.float32(0))
    gide = jnp.dot(gidf, ef, preferred_element_type=jnp.float32)
    posi = lax.broadcasted_iota(jnp.int32, (QB, K * G), 1)
    cidx = gide.astype(jnp.int32) * G + posi % G
    acc0 = jnp.zeros((QB, K), jnp.int32)
    lane = lax.broadcasted_iota(jnp.int32, acc0.shape, 1)

    def step(tstep, carry):
        d2f, acc = carry
        m = jnp.min(d2f, axis=1, keepdims=True)
        j = jnp.min(jnp.where(d2f <= m, cidx, jnp.int32(2 ** 30)),
                    axis=1, keepdims=True)
        d2f = jnp.where(cidx == j, jnp.float32(3.0e38), d2f)
        acc = jnp.where(lane == tstep, j, acc)
        return d2f, acc

    _, acc = lax.fori_loop(0, K, step, (d2c, acc0))
    idx_ref[...] = acc


def _run_knn_select(d2r, gid2):
    QB = 128
    n = gid2.shape[0]
    return pl.pallas_call(
        _knn_select_body,
        grid=(n // QB,),
        in_specs=[
            pl.BlockSpec((QB, K * G), lambda i: (i, 0)),
            pl.BlockSpec((QB, K), lambda i: (i, 0)),
        ],
        out_specs=pl.BlockSpec((QB, K), lambda i: (i, 0)),
        out_shape=jax.ShapeDtypeStruct((n, K), jnp.int32),
        compiler_params=pltpu.CompilerParams(
            dimension_semantics=("parallel",)),
    )(d2r, gid2)


_NC, _NS = 2, 16
_NW = _NC * _NS
_CHUNK = 128


def _make_sc_gather(R, Dw):
    rows_per_w = R // _NW
    nchunk = rows_per_w // _CHUNK
    mesh = plsc.VectorSubcoreMesh(core_axis_name="c", subcore_axis_name="s")

    @functools.partial(
        pl.kernel,
        mesh=mesh,
        out_type=jax.ShapeDtypeStruct((R, Dw), jnp.float32),
        scratch_types=[
            pltpu.VMEM((_CHUNK,), jnp.int32),
            pltpu.VMEM((_CHUNK,), jnp.int32),
            pltpu.VMEM((_CHUNK, Dw), jnp.float32),
            pltpu.VMEM((_CHUNK, Dw), jnp.float32),
            pltpu.SemaphoreType.DMA,
            pltpu.SemaphoreType.DMA,
        ],
    )
    def gather_k(table_hbm, idx_hbm, out_hbm, idx_a, idx_b, rows_a, rows_b,
                 sem_a, sem_b):
        wid = lax.axis_index("s") * _NC + lax.axis_index("c")
        base0 = wid * nchunk * _CHUNK
        pltpu.sync_copy(idx_hbm.at[pl.ds(base0, _CHUNK)], idx_a)
        pltpu.make_async_copy(table_hbm.at[idx_a], rows_a, sem_a).start()

        def body(i, _):
            even = lax.rem(i, 2) == 0
            base_n = base0 + (i + 1) * _CHUNK

            def do(idx_c, rows_c, sem_c, idx_n, rows_n, sem_n):
                @pl.when(i + 1 < nchunk)
                def _():
                    pltpu.sync_copy(idx_hbm.at[pl.ds(base_n, _CHUNK)], idx_n)
                    pltpu.make_async_copy(table_hbm.at[idx_n], rows_n,
                                          sem_n).start()

                pltpu.make_async_copy(table_hbm.at[idx_c], rows_c, sem_c).wait()
                pltpu.sync_copy(
                    rows_c, out_hbm.at[pl.ds(base0 + i * _CHUNK, _CHUNK)])

            @pl.when(even)
            def _():
                do(idx_a, rows_a, sem_a, idx_b, rows_b, sem_b)

            @pl.when(jnp.logical_not(even))
            def _():
                do(idx_b, rows_b, sem_b, idx_a, rows_a, sem_a)

            return 0

        lax.fori_loop(0, nchunk, body, 0)

    return gather_k


def _attn_body(g_ref, q_ref, tab_ref, wp2_ref, b2_ref, b1_ref,
               wproj_ref, bproj_ref, o_ref):
    QD = q_ref.shape[0]
    g = g_ref[...]
    k_nei = g[:, 0:D]
    v_nei = g[:, D:2 * D]
    t_nei = g[:, 2 * D:3 * D].reshape(QD, K, D)
    tq = tab_ref[:, 2 * D:3 * D]
    hid = jnp.maximum(t_nei - tq[:, None, :] + b1_ref[...], 0.0)
    pos = jnp.dot(hid.reshape(QD * K, D), wp2_ref[...],
                  preferred_element_type=jnp.float32) + b2_ref[...]
    q = q_ref[...]
    s = (k_nei + pos).reshape(QD, K, D) * q[:, None, :]
    ci = lax.broadcasted_iota(jnp.int32, (D, H), 0)
    hi = lax.broadcasted_iota(jnp.int32, (D, H), 1)
    eh = jnp.where(ci // DH == hi, jnp.float32(1), jnp.float32(0))
    logits = jnp.dot(s.reshape(QD * K, D), eh,
                     preferred_element_type=jnp.float32)
    logits = logits.reshape(QD, K, H) * SCALE
    m = jnp.max(logits, axis=1, keepdims=True)
    e = jnp.exp(logits - m)
    a = e / jnp.sum(e, axis=1, keepdims=True)
    cit = lax.broadcasted_iota(jnp.int32, (H, D), 0)
    hit = lax.broadcasted_iota(jnp.int32, (H, D), 1)
    eht = jnp.where(hit // DH == cit, jnp.float32(1), jnp.float32(0))
    a128 = jnp.dot(a.reshape(QD * K, H), eht,
                   preferred_element_type=jnp.float32).reshape(QD, K, D)
    o = jnp.sum(a128 * ((v_nei + pos).reshape(QD, K, D)), axis=1)
    o_ref[...] = jnp.dot(o, wproj_ref[...],
                         preferred_element_type=jnp.float32) + bproj_ref[...]


def _run_attn(g, q2, tab, Wp2, b2r, b1r, Wproj, bprojr):
    QD = 128
    n = q2.shape[0]
    return pl.pallas_call(
        _attn_body,
        grid=(n // QD,),
        in_specs=[
            pl.BlockSpec((QD * K, 3 * D), lambda i: (i, 0)),
            pl.BlockSpec((QD, D), lambda i: (i, 0)),
            pl.BlockSpec((QD, 3 * D), lambda i: (i, 0)),
            pl.BlockSpec((D, D), lambda i: (0, 0)),
            pl.BlockSpec((1, D), lambda i: (0, 0)),
            pl.BlockSpec((1, D), lambda i: (0, 0)),
            pl.BlockSpec((D, D), lambda i: (0, 0)),
            pl.BlockSpec((1, D), lambda i: (0, 0)),
        ],
        out_specs=pl.BlockSpec((QD, D), lambda i: (i, 0)),
        out_shape=jax.ShapeDtypeStruct((n, D), jnp.float32),
        compiler_params=pltpu.CompilerParams(
            dimension_semantics=("parallel",)),
    )(g, q2, tab, Wp2, b2r, b1r, Wproj, bprojr)


def kernel(xyz, feats, Wq, Wk, Wv, Wproj, bproj, Wp1, bp1, Wp2, bp2):
    pad = jnp.zeros((B, P, 5), xyz.dtype)
    xyz8 = jnp.concatenate([xyz, pad], axis=-1)
    xyzT8 = jnp.swapaxes(xyz8, 1, 2)
    Wp1p = jnp.concatenate([Wp1, jnp.zeros((5, D), Wp1.dtype)], axis=0)

    feats2 = feats.reshape(B * P, D)
    xyz8f = xyz8.reshape(B * P, 8)

    q2, tab = _run_proj(feats2, xyz8f, Wq, Wk, Wv, Wp1p)
    gid = _run_knn_groups(xyz8, xyzT8)
    xg = xyz8.reshape(B, NG, G, 8)
    gt = jnp.stack([xg[..., 0], xg[..., 1], xg[..., 2],
                    jnp.sum(xg * xg, axis=-1)], axis=2).reshape(B * NG, 4 * G)
    g1 = _make_sc_gather(B * P * K, 4 * G)(gt, gid.reshape(B * P * K))
    d2r = _run_knn_dist(g1, xyz8f).reshape(B * P, K * G)
    idx = _run_knn_select(d2r, gid.reshape(B * P, K))
    g = _make_sc_gather(B * P * K, 3 * D)(tab, idx.reshape(B * P * K))
    out = _run_attn(g, q2, tab, Wp2, bp2.reshape(1, D), bp1.reshape(1, D),
                    Wproj, bproj.reshape(1, D))
    return out.reshape(B, P, D)
